# baseline (speedup 1.0000x reference)
import jax
import jax.numpy as jnp
from jax.experimental import pallas as pl
from jax.experimental.pallas import tpu as pltpu

_N_NEIGHBORS = 15
_NUM_CLASSES = 10
_Q = 1024
_QT = 128
_D = 16
_KBLK = 256
_INF = float("inf")


def _knn_block_kernel(x_ref, tx_ref, code_ref, probs_ref, preds_ref,
                      run_d_ref, run_c_ref, d2_ref):
    b = pl.program_id(0)
    nb = pl.num_programs(0)
    qt = pl.program_id(1)
    qrows = pl.ds(qt * _QT, _QT)

    @pl.when(b == 0)
    def _init():
        run_d_ref[qrows, :] = jnp.concatenate(
            [jnp.full((_QT, 15), _INF, jnp.float32),
             jnp.full((_QT, 1), -_INF, jnp.float32)], axis=1)
        run_c_ref[qrows, :] = jnp.zeros((_QT, 16), jnp.float32)

    xq = x_ref[qrows, :]
    tx = tx_ref[...]
    code = code_ref[0]

    ksq = jnp.sum(tx * tx, axis=1)[None, :]
    qk = jax.lax.dot_general(
        xq, tx, (((1,), (1,)), ((), ())),
        preferred_element_type=jnp.float32)
    d2_ref[...] = ksq - 2.0 * qk

    iota16 = jax.lax.broadcasted_iota(jnp.int32, (_QT, 16), 1).astype(jnp.float32)
    big = jnp.float32(1e9)

    def _extract(_, carry):
        rd, rc = carry
        d2 = d2_ref[...]
        m = jnp.min(d2, axis=1, keepdims=True)
        w = jnp.max(rd, axis=1, keepdims=True)
        imp = m < w
        cm = jnp.min(jnp.where(d2 == m, code, big),
                     axis=1, keepdims=True)
        d2_ref[...] = jnp.where((code == cm) & imp, _INF, d2)
        wio = jnp.where(rd == w, iota16, big)
        wsel = (wio == jnp.min(wio, axis=1, keepdims=True)) & imp
        rd = jnp.where(wsel, m, rd)
        rc = jnp.where(wsel, cm, rc)
        return rd, rc

    rd, rc = jax.lax.fori_loop(
        0, _N_NEIGHBORS, _extract,
        (run_d_ref[qrows, :], run_c_ref[qrows, :]), unroll=1)

    run_d_ref[qrows, :] = rd
    run_c_ref[qrows, :] = rc

    @pl.when(b == nb - 1)
    def _finalize():
        valid = jnp.isfinite(rd)
        lab = rc - 16.0 * jnp.floor(rc * (1.0 / 16.0))
        pm = jnp.full((_QT, 1), -1.0, jnp.float32)
        pc = jnp.zeros((_QT, 1), jnp.float32)
        for c in range(_NUM_CLASSES):
            hit = (lab == jnp.float32(c)) & valid
            cnt = jnp.sum(hit.astype(jnp.float32), axis=1, keepdims=True)
            probs_ref[qrows, c:c + 1] = cnt * jnp.float32(1.0 / _N_NEIGHBORS)
            upd = cnt > pm
            pm = jnp.where(upd, cnt, pm)
            pc = jnp.where(upd, jnp.float32(c), pc)
        probs_ref[qrows, _NUM_CLASSES:] = jnp.zeros(
            (_QT, 16 - _NUM_CLASSES), jnp.float32)
        preds_ref[qrows, :] = pc.astype(jnp.int32)


def kernel(x, train_x, train_y):
    n = train_x.shape[0]
    nb = (n + _KBLK - 1) // _KBLK
    npad = nb * _KBLK - n
    tx = jnp.pad(train_x, ((0, npad), (0, 0)), constant_values=1e3)
    lane = jnp.tile(jnp.arange(_KBLK, dtype=jnp.int32), nb)
    code = (lane * 16 + jnp.pad(train_y, (0, npad))).astype(jnp.float32)
    code = code.reshape(nb, 1, _KBLK)

    probs16, preds2 = pl.pallas_call(
        _knn_block_kernel,
        grid=(nb, _Q // _QT),
        in_specs=[
            pl.BlockSpec((_Q, _D), lambda b, q: (0, 0)),
            pl.BlockSpec((_KBLK, _D), lambda b, q: (b, 0)),
            pl.BlockSpec((1, 1, _KBLK), lambda b, q: (b, 0, 0)),
        ],
        out_specs=(
            pl.BlockSpec((_Q, 16), lambda b, q: (0, 0)),
            pl.BlockSpec((_Q, 1), lambda b, q: (0, 0)),
        ),
        out_shape=(
            jax.ShapeDtypeStruct((_Q, 16), jnp.float32),
            jax.ShapeDtypeStruct((_Q, 1), jnp.int32),
        ),
        scratch_shapes=[
            pltpu.VMEM((_Q, 16), jnp.float32),
            pltpu.VMEM((_Q, 16), jnp.float32),
            pltpu.VMEM((_QT, _KBLK), jnp.float32),
        ],
    )(x, tx, code)

    return preds2[:, 0], probs16[:, :_NUM_CLASSES]


# speedup vs baseline: 1.1511x; 1.1511x over previous
import jax
import jax.numpy as jnp
from jax.experimental import pallas as pl
from jax.experimental.pallas import tpu as pltpu

_N_NEIGHBORS = 15
_NUM_CLASSES = 10
_Q = 1024
_QT = 128
_D = 16
_KBLK = 512
_INF = float("inf")


def _knn_block_kernel(x_ref, tx_ref, code_ref, probs_ref, preds_ref,
                      run_d_ref, run_c_ref, d2_ref):
    b = pl.program_id(0)
    nb = pl.num_programs(0)
    qt = pl.program_id(1)
    qrows = pl.ds(qt * _QT, _QT)

    @pl.when(b == 0)
    def _init():
        run_d_ref[qrows, :] = jnp.concatenate(
            [jnp.full((_QT, 15), _INF, jnp.float32),
             jnp.full((_QT, 1), -_INF, jnp.float32)], axis=1)
        run_c_ref[qrows, :] = jnp.zeros((_QT, 16), jnp.float32)

    xq = x_ref[qrows, :]
    tx = tx_ref[...]
    code = code_ref[0]

    ksq = jnp.sum(tx * tx, axis=1)[None, :]
    qk = jax.lax.dot_general(
        xq, tx, (((1,), (1,)), ((), ())),
        preferred_element_type=jnp.float32)
    d2_ref[...] = ksq - 2.0 * qk

    iota16 = jax.lax.broadcasted_iota(jnp.int32, (_QT, 16), 1).astype(jnp.float32)
    big = jnp.float32(1e9)

    def _extract(_, carry):
        rd, rc = carry
        d2 = d2_ref[...]
        m = jnp.min(d2, axis=1, keepdims=True)
        w = jnp.max(rd, axis=1, keepdims=True)
        imp = m < w
        cm = jnp.min(jnp.where(d2 == m, code, big),
                     axis=1, keepdims=True)
        d2_ref[...] = jnp.where((code == cm) & imp, _INF, d2)
        wio = jnp.where(rd == w, iota16, big)
        wsel = (wio == jnp.min(wio, axis=1, keepdims=True)) & imp
        rd = jnp.where(wsel, m, rd)
        rc = jnp.where(wsel, cm, rc)
        return rd, rc

    rd, rc = jax.lax.fori_loop(
        0, _N_NEIGHBORS, _extract,
        (run_d_ref[qrows, :], run_c_ref[qrows, :]), unroll=1)

    run_d_ref[qrows, :] = rd
    run_c_ref[qrows, :] = rc

    @pl.when(b == nb - 1)
    def _finalize():
        valid = jnp.isfinite(rd)
        lab = rc - 16.0 * jnp.floor(rc * (1.0 / 16.0))
        pm = jnp.full((_QT, 1), -1.0, jnp.float32)
        pc = jnp.zeros((_QT, 1), jnp.float32)
        for c in range(_NUM_CLASSES):
            hit = (lab == jnp.float32(c)) & valid
            cnt = jnp.sum(hit.astype(jnp.float32), axis=1, keepdims=True)
            probs_ref[qrows, c:c + 1] = cnt * jnp.float32(1.0 / _N_NEIGHBORS)
            upd = cnt > pm
            pm = jnp.where(upd, cnt, pm)
            pc = jnp.where(upd, jnp.float32(c), pc)
        probs_ref[qrows, _NUM_CLASSES:] = jnp.zeros(
            (_QT, 16 - _NUM_CLASSES), jnp.float32)
        preds_ref[qrows, :] = pc.astype(jnp.int32)


def kernel(x, train_x, train_y):
    n = train_x.shape[0]
    nb = (n + _KBLK - 1) // _KBLK
    npad = nb * _KBLK - n
    tx = jnp.pad(train_x, ((0, npad), (0, 0)), constant_values=1e3)
    lane = jnp.tile(jnp.arange(_KBLK, dtype=jnp.int32), nb)
    code = (lane * 16 + jnp.pad(train_y, (0, npad))).astype(jnp.float32)
    code = code.reshape(nb, 1, _KBLK)

    probs16, preds2 = pl.pallas_call(
        _knn_block_kernel,
        grid=(nb, _Q // _QT),
        in_specs=[
            pl.BlockSpec((_Q, _D), lambda b, q: (0, 0)),
            pl.BlockSpec((_KBLK, _D), lambda b, q: (b, 0)),
            pl.BlockSpec((1, 1, _KBLK), lambda b, q: (b, 0, 0)),
        ],
        out_specs=(
            pl.BlockSpec((_Q, 16), lambda b, q: (0, 0)),
            pl.BlockSpec((_Q, 1), lambda b, q: (0, 0)),
        ),
        out_shape=(
            jax.ShapeDtypeStruct((_Q, 16), jnp.float32),
            jax.ShapeDtypeStruct((_Q, 1), jnp.int32),
        ),
        scratch_shapes=[
            pltpu.VMEM((_Q, 16), jnp.float32),
            pltpu.VMEM((_Q, 16), jnp.float32),
            pltpu.VMEM((_QT, _KBLK), jnp.float32),
        ],
    )(x, tx, code)

    return preds2[:, 0], probs16[:, :_NUM_CLASSES]
